# baseline (speedup 1.0000x reference)
import jax
import jax.numpy as jnp
from jax import lax
from jax.experimental import pallas as pl
from jax.experimental.pallas import tpu as pltpu

N = 10000
N_PAD = 10240
D = 128
NC = 8
NB = 2
GW = 16
E_PAD = 331776
EROWS = E_PAD // (NC * GW)
F32 = jnp.float32


def _proj0(x_pad, wl, wr):
    def body(x_ref, wl_ref, wr_ref, xl_ref, xr_ref):
        xb = x_ref[...]
        xl_ref[...] = jnp.dot(xb, wl_ref[...], preferred_element_type=F32)
        xr_ref[...] = jnp.dot(xb, wr_ref[...], preferred_element_type=F32)

    return pl.pallas_call(
        body,
        grid=(N_PAD // 256,),
        in_specs=[
            pl.BlockSpec((256, D), lambda i: (i, 0)),
            pl.BlockSpec((D, D), lambda i: (0, 0)),
            pl.BlockSpec((D, D), lambda i: (0, 0)),
        ],
        out_specs=[pl.BlockSpec((256, D), lambda i: (i, 0))] * 2,
        out_shape=[jax.ShapeDtypeStruct((N_PAD, D), F32)] * 2,
    )(x_pad, wl, wr)


def _proj_ep(num, den, b2d, wl, wr):
    def body(num_ref, den_ref, b_ref, wl_ref, wr_ref, xl_ref, xr_ref):
        nsum = jnp.sum(num_ref[...], axis=(0, 1))
        dsum = jnp.sum(den_ref[...], axis=(0, 2))
        h = nsum / (dsum[:, None] + 1e-16) + b_ref[...]
        h = jnp.maximum(h, 0.0)
        xl_ref[...] = jnp.dot(h, wl_ref[...], preferred_element_type=F32)
        xr_ref[...] = jnp.dot(h, wr_ref[...], preferred_element_type=F32)

    return pl.pallas_call(
        body,
        grid=(N_PAD // 256,),
        in_specs=[
            pl.BlockSpec((NC, NB, 256, D), lambda i: (0, 0, i, 0)),
            pl.BlockSpec((NC, 256, 16), lambda i: (0, i, 0)),
            pl.BlockSpec((1, D), lambda i: (0, 0)),
            pl.BlockSpec((D, D), lambda i: (0, 0)),
            pl.BlockSpec((D, D), lambda i: (0, 0)),
        ],
        out_specs=[pl.BlockSpec((256, D), lambda i: (i, 0))] * 2,
        out_shape=[jax.ShapeDtypeStruct((N_PAD, D), F32)] * 2,
    )(num, den, b2d, wl, wr)


def _final(num, den, b2d):
    def body(num_ref, den_ref, b_ref, o_ref):
        nsum = jnp.sum(num_ref[...], axis=(0, 1))
        dsum = jnp.sum(den_ref[...], axis=(0, 2))
        h = nsum / (dsum[:, None] + 1e-16) + b_ref[...]
        h = jnp.maximum(h, 0.0)
        o_ref[...] = jax.nn.sigmoid(h)

    return pl.pallas_call(
        body,
        grid=(N_PAD // 256,),
        in_specs=[
            pl.BlockSpec((NC, NB, 256, D), lambda i: (0, 0, i, 0)),
            pl.BlockSpec((NC, 256, 16), lambda i: (0, i, 0)),
            pl.BlockSpec((1, D), lambda i: (0, 0)),
        ],
        out_specs=pl.BlockSpec((256, D), lambda i: (i, 0)),
        out_shape=jax.ShapeDtypeStruct((N_PAD, D), F32),
    )(num, den, b2d)


def _edge(xl, xr, att2d, comb):
    def body(xl_ref, xr_ref, att_ref, comb_ref, num_ref, den_ref):
        num_ref[...] = jnp.zeros((1, NB, N_PAD, D), F32)
        den_ref[...] = jnp.zeros((1, N_PAD, 16), F32)
        attv = att_ref[...]
        dmask = (lax.broadcasted_iota(jnp.int32, (1, 16), 1) == 0
                 ).astype(F32)

        def row_body(r, carry):
            rowv = comb_ref[0, pl.ds(r, 1), :].reshape(GW)
            for c in range(GW):
                k = c % NB
                p = rowv[c]
                s = jnp.bitwise_and(p, 16383)
                d = jnp.right_shift(p, 14)
                a = xl_ref[pl.ds(s, 1), :]
                bb = xr_ref[pl.ds(d, 1), :]
                e = a + bb
                lr = jnp.where(e >= 0.0, e, 0.2 * e)
                w = jnp.exp(jnp.sum(lr * attv))
                cur = num_ref[0, k, pl.ds(d, 1), :]
                num_ref[0, k, pl.ds(d, 1), :] = cur + w * a
                curd = den_ref[0, pl.ds(d, 1), :]
                den_ref[0, pl.ds(d, 1), :] = curd + w * dmask
            return carry

        lax.fori_loop(0, EROWS, row_body, 0)

    return pl.pallas_call(
        body,
        grid=(NC,),
        in_specs=[
            pl.BlockSpec((N_PAD, D), lambda i: (0, 0)),
            pl.BlockSpec((N_PAD, D), lambda i: (0, 0)),
            pl.BlockSpec((1, D), lambda i: (0, 0)),
            pl.BlockSpec((1, EROWS, GW), lambda i: (i, 0, 0)),
        ],
        out_specs=[
            pl.BlockSpec((1, NB, N_PAD, D), lambda i: (i, 0, 0, 0)),
            pl.BlockSpec((1, N_PAD, 16), lambda i: (i, 0, 0)),
        ],
        out_shape=[
            jax.ShapeDtypeStruct((NC, NB, N_PAD, D), F32),
            jax.ShapeDtypeStruct((NC, N_PAD, 16), F32),
        ],
        compiler_params=pltpu.CompilerParams(
            dimension_semantics=("parallel",)),
    )(xl, xr, att2d, comb)


def kernel(x, edge_index, Wl0, Wr0, att0, b0, Wl1, Wr1, att1, b1,
           Wl2, Wr2, att2, b2):
    n = x.shape[0]
    loop = jnp.arange(n, dtype=edge_index.dtype)
    src = jnp.concatenate([edge_index[0], loop])
    dst = jnp.concatenate([edge_index[1], loop])
    e = src.shape[0]
    pad = E_PAD - e
    src = jnp.concatenate([src, jnp.zeros((pad,), src.dtype)])
    dst = jnp.concatenate([dst, jnp.full((pad,), n, dst.dtype)])
    src2 = src.astype(jnp.int32)
    dst2 = dst.astype(jnp.int32)
    comb = jnp.bitwise_or(jnp.left_shift(dst2, 14), src2)
    comb = comb.reshape(NC, EROWS, GW)

    x_pad = jnp.pad(x, ((0, N_PAD - n), (0, 0)))
    a0 = att0.reshape(1, D)
    a1 = att1.reshape(1, D)
    a2 = att2.reshape(1, D)
    b0_2d = b0.reshape(1, D)
    b1_2d = b1.reshape(1, D)
    b2_2d = b2.reshape(1, D)

    xl, xr = _proj0(x_pad, Wl0, Wr0)
    num, den = _edge(xl, xr, a0, comb)
    xl, xr = _proj_ep(num, den, b0_2d, Wl1, Wr1)
    num, den = _edge(xl, xr, a1, comb)
    xl, xr = _proj_ep(num, den, b1_2d, Wl2, Wr2)
    num, den = _edge(xl, xr, a2, comb)
    return _final(num, den, b2_2d)[:n]


# speedup vs baseline: 3.3319x; 3.3319x over previous
import jax
import jax.numpy as jnp
from jax import lax
from jax.experimental import pallas as pl
from jax.experimental.pallas import tpu as pltpu

N = 10000
N_PAD = 10240
D = 128
NC = 8
NB = 2
GW = 16
E_PAD = 331776
EROWS = E_PAD // (NC * GW)
RCH = 144
NRC = EROWS // RCH
F32 = jnp.float32


def _proj0(x_pad, wl, wr):
    def body(x_ref, wl_ref, wr_ref, xl_ref, xr_ref):
        xb = x_ref[...]
        xl_ref[...] = jnp.dot(xb, wl_ref[...], preferred_element_type=F32)
        xr_ref[...] = jnp.dot(xb, wr_ref[...], preferred_element_type=F32)

    return pl.pallas_call(
        body,
        grid=(N_PAD // 256,),
        in_specs=[
            pl.BlockSpec((256, D), lambda i: (i, 0)),
            pl.BlockSpec((D, D), lambda i: (0, 0)),
            pl.BlockSpec((D, D), lambda i: (0, 0)),
        ],
        out_specs=[pl.BlockSpec((256, D), lambda i: (i, 0))] * 2,
        out_shape=[jax.ShapeDtypeStruct((N_PAD, D), F32)] * 2,
    )(x_pad, wl, wr)


def _proj_ep(num, den, b2d, wl, wr):
    def body(num_ref, den_ref, b_ref, wl_ref, wr_ref, xl_ref, xr_ref):
        nsum = jnp.sum(num_ref[...], axis=(0, 1))
        dsum = jnp.sum(den_ref[...], axis=(0, 2))
        h = nsum / (dsum[:, None] + 1e-16) + b_ref[...]
        h = jnp.maximum(h, 0.0)
        xl_ref[...] = jnp.dot(h, wl_ref[...], preferred_element_type=F32)
        xr_ref[...] = jnp.dot(h, wr_ref[...], preferred_element_type=F32)

    return pl.pallas_call(
        body,
        grid=(N_PAD // 256,),
        in_specs=[
            pl.BlockSpec((NC, NB, 256, D), lambda i: (0, 0, i, 0)),
            pl.BlockSpec((NC, 256, 16), lambda i: (0, i, 0)),
            pl.BlockSpec((1, D), lambda i: (0, 0)),
            pl.BlockSpec((D, D), lambda i: (0, 0)),
            pl.BlockSpec((D, D), lambda i: (0, 0)),
        ],
        out_specs=[pl.BlockSpec((256, D), lambda i: (i, 0))] * 2,
        out_shape=[jax.ShapeDtypeStruct((N_PAD, D), F32)] * 2,
    )(num, den, b2d, wl, wr)


def _final(num, den, b2d):
    def body(num_ref, den_ref, b_ref, o_ref):
        nsum = jnp.sum(num_ref[...], axis=(0, 1))
        dsum = jnp.sum(den_ref[...], axis=(0, 2))
        h = nsum / (dsum[:, None] + 1e-16) + b_ref[...]
        h = jnp.maximum(h, 0.0)
        o_ref[...] = jax.nn.sigmoid(h)

    return pl.pallas_call(
        body,
        grid=(N_PAD // 256,),
        in_specs=[
            pl.BlockSpec((NC, NB, 256, D), lambda i: (0, 0, i, 0)),
            pl.BlockSpec((NC, 256, 16), lambda i: (0, i, 0)),
            pl.BlockSpec((1, D), lambda i: (0, 0)),
        ],
        out_specs=pl.BlockSpec((256, D), lambda i: (i, 0)),
        out_shape=jax.ShapeDtypeStruct((N_PAD, D), F32),
    )(num, den, b2d)


def _edge(xl, xr, att2d, comb):
    def body(xl_ref, xr_ref, att_ref, comb_ref, num_ref, den_ref):
        @pl.when(pl.program_id(1) == 0)
        def _init():
            num_ref[...] = jnp.zeros((1, NB, N_PAD, D), F32)
            den_ref[...] = jnp.zeros((1, N_PAD, 16), F32)

        attv = att_ref[...]
        dmask = (lax.broadcasted_iota(jnp.int32, (1, 16), 1) == 0
                 ).astype(F32)

        def row_body(r, carry):
            for c in range(GW):
                k = c % NB
                p = comb_ref[0, r, c]
                s = jnp.bitwise_and(p, 16383)
                d = jnp.right_shift(p, 14)
                a = xl_ref[pl.ds(s, 1), :]
                bb = xr_ref[pl.ds(d, 1), :]
                e = a + bb
                lr = jnp.where(e >= 0.0, e, 0.2 * e)
                w = jnp.exp(jnp.sum(lr * attv))
                cur = num_ref[0, k, pl.ds(d, 1), :]
                num_ref[0, k, pl.ds(d, 1), :] = cur + w * a
                curd = den_ref[0, pl.ds(d, 1), :]
                den_ref[0, pl.ds(d, 1), :] = curd + w * dmask
            return carry

        lax.fori_loop(0, RCH, row_body, 0)

    return pl.pallas_call(
        body,
        grid=(NC, NRC),
        in_specs=[
            pl.BlockSpec((N_PAD, D), lambda i, j: (0, 0)),
            pl.BlockSpec((N_PAD, D), lambda i, j: (0, 0)),
            pl.BlockSpec((1, D), lambda i, j: (0, 0)),
            pl.BlockSpec((1, RCH, GW), lambda i, j: (i, j, 0),
                         memory_space=pltpu.SMEM),
        ],
        out_specs=[
            pl.BlockSpec((1, NB, N_PAD, D), lambda i, j: (i, 0, 0, 0)),
            pl.BlockSpec((1, N_PAD, 16), lambda i, j: (i, 0, 0)),
        ],
        out_shape=[
            jax.ShapeDtypeStruct((NC, NB, N_PAD, D), F32),
            jax.ShapeDtypeStruct((NC, N_PAD, 16), F32),
        ],
        compiler_params=pltpu.CompilerParams(
            dimension_semantics=("parallel", "arbitrary")),
    )(xl, xr, att2d, comb)


def kernel(x, edge_index, Wl0, Wr0, att0, b0, Wl1, Wr1, att1, b1,
           Wl2, Wr2, att2, b2):
    n = x.shape[0]
    loop = jnp.arange(n, dtype=edge_index.dtype)
    src = jnp.concatenate([edge_index[0], loop])
    dst = jnp.concatenate([edge_index[1], loop])
    e = src.shape[0]
    pad = E_PAD - e
    src = jnp.concatenate([src, jnp.zeros((pad,), src.dtype)])
    dst = jnp.concatenate([dst, jnp.full((pad,), n, dst.dtype)])
    src2 = src.astype(jnp.int32)
    dst2 = dst.astype(jnp.int32)
    comb = jnp.bitwise_or(jnp.left_shift(dst2, 14), src2)
    comb = comb.reshape(NC, EROWS, GW)

    x_pad = jnp.pad(x, ((0, N_PAD - n), (0, 0)))
    a0 = att0.reshape(1, D)
    a1 = att1.reshape(1, D)
    a2 = att2.reshape(1, D)
    b0_2d = b0.reshape(1, D)
    b1_2d = b1.reshape(1, D)
    b2_2d = b2.reshape(1, D)

    xl, xr = _proj0(x_pad, Wl0, Wr0)
    num, den = _edge(xl, xr, a0, comb)
    xl, xr = _proj_ep(num, den, b0_2d, Wl1, Wr1)
    num, den = _edge(xl, xr, a1, comb)
    xl, xr = _proj_ep(num, den, b1_2d, Wl2, Wr2)
    num, den = _edge(xl, xr, a2, comb)
    return _final(num, den, b2_2d)[:n]
